# baseline (speedup 1.0000x reference)
import functools

import jax
import jax.numpy as jnp
from jax.experimental import pallas as pl
from jax.experimental.pallas import tpu as pltpu

N_NODES = 50000
N_EDGES = 800000
ET = 2000
NT = 2000


def _sp(x):
    return jax.nn.softplus(x)


def _wb(layers):
    flat = []
    for l in layers:
        flat.append(l['W'])
        flat.append(l['b'].reshape(1, -1))
    return flat


def _chain(h, refs, start, n_layers, activate_last):
    for i in range(n_layers):
        W = refs[start + 2 * i][...]
        b = refs[start + 2 * i + 1][...]
        h = jnp.dot(h, W, preferred_element_type=jnp.float32) + b
        if i < n_layers - 1 or activate_last:
            h = _sp(h)
    return h


def _node_enc(node_type, emb, enc_layers):
    flat = _wb(enc_layers)
    nl = len(enc_layers)
    ntypes = emb.shape[0]

    def body(t_ref, emb_ref, *refs):
        out_ref = refs[-1]
        t = t_ref[...]
        onehot = (t == jax.lax.broadcasted_iota(jnp.int32, (1, ntypes), 1))
        v0 = jnp.dot(onehot.astype(jnp.float32), emb_ref[...],
                     preferred_element_type=jnp.float32)
        out_ref[...] = _chain(v0, refs, 0, nl, True)

    grid = N_NODES // NT
    in_specs = [pl.BlockSpec((NT, 1), lambda i: (i, 0)),
                pl.BlockSpec(emb.shape, lambda i: (0, 0))]
    for w in flat:
        in_specs.append(pl.BlockSpec(w.shape, lambda i: (0, 0)))
    return pl.pallas_call(
        body, grid=(grid,), in_specs=in_specs,
        out_specs=pl.BlockSpec((NT, 32), lambda i: (i, 0)),
        out_shape=jax.ShapeDtypeStruct((N_NODES, 32), jnp.float32),
    )(node_type.reshape(N_NODES, 1), emb, *flat)


def _rowwise_mlp(x, layers, tile, activate_last=True):
    flat = _wb(layers)
    nl = len(layers)
    rows, din = x.shape
    dout = layers[-1]['W'].shape[1]

    def body(x_ref, *refs):
        out_ref = refs[-1]
        out_ref[...] = _chain(x_ref[...], refs, 0, nl, activate_last)

    in_specs = [pl.BlockSpec((tile, din), lambda i: (i, 0))]
    for w in flat:
        in_specs.append(pl.BlockSpec(w.shape, lambda i: (0, 0)))
    return pl.pallas_call(
        body, grid=(rows // tile,), in_specs=in_specs,
        out_specs=pl.BlockSpec((tile, dout), lambda i: (i, 0)),
        out_shape=jax.ShapeDtypeStruct((rows, dout), jnp.float32),
    )(x, *flat)


def _small_mlp(x, layers, activate_last=True):
    flat = _wb(layers)
    nl = len(layers)
    dout = layers[-1]['W'].shape[1]

    def body(x_ref, *refs):
        out_ref = refs[-1]
        out_ref[...] = _chain(x_ref[...], refs, 0, nl, activate_last)

    return pl.pallas_call(
        body,
        out_shape=jax.ShapeDtypeStruct((1, dout), jnp.float32),
    )(x, *flat)


def _edge_kernel(e_in, gs, gd, u_f, ef_layers, conv_layers):
    W1, b1 = conv_layers[0]['W'], conv_layers[0]['b'].reshape(1, -1)
    W1s, W1d, W1e, W1u = W1[0:32], W1[32:64], W1[64:96], W1[96:128]
    rest = _wb(conv_layers[1:])
    nrest = len(conv_layers) - 1
    ef_flat = _wb(ef_layers)
    nef = len(ef_layers)

    def body(e_ref, gs_ref, gd_ref, u_ref, *refs):
        enew_ref, eout_ref, acc_ref = refs[-3], refs[-2], refs[-1]
        e0 = e_ref[...]
        if nef:
            e_f = _chain(e0, refs, 0, nef, True)
        else:
            e_f = e0
        k = 2 * nef
        W1s_, W1d_, W1e_, W1u_, b1_ = (refs[k][...], refs[k + 1][...],
                                       refs[k + 2][...], refs[k + 3][...],
                                       refs[k + 4][...])
        uvec = jnp.dot(u_ref[...], W1u_, preferred_element_type=jnp.float32)
        h = (jnp.dot(gs_ref[...], W1s_, preferred_element_type=jnp.float32)
             + jnp.dot(gd_ref[...], W1d_, preferred_element_type=jnp.float32)
             + jnp.dot(e_f, W1e_, preferred_element_type=jnp.float32)
             + uvec + b1_)
        h = _sp(h)
        e_new = _chain(h, refs, k + 5, nrest, True)
        enew_ref[...] = e_new
        eout_ref[...] = e_new + e0
        i = pl.program_id(0)

        @pl.when(i == 0)
        def _():
            acc_ref[...] = jnp.zeros_like(acc_ref)

        acc_ref[...] += jnp.sum(e_new, axis=0, keepdims=True)

    flat_in = [e_in, gs, gd, u_f] + ef_flat + [W1s, W1d, W1e, W1u, b1] + rest
    in_specs = ([pl.BlockSpec((ET, 32), lambda i: (i, 0))] * 3
                + [pl.BlockSpec((1, 32), lambda i: (0, 0))])
    for w in flat_in[4:]:
        in_specs.append(pl.BlockSpec(w.shape, lambda i: (0, 0)))
    return pl.pallas_call(
        body, grid=(N_EDGES // ET,), in_specs=in_specs,
        out_specs=[pl.BlockSpec((ET, 32), lambda i: (i, 0)),
                   pl.BlockSpec((ET, 32), lambda i: (i, 0)),
                   pl.BlockSpec((1, 32), lambda i: (0, 0))],
        out_shape=[jax.ShapeDtypeStruct((N_EDGES, 32), jnp.float32),
                   jax.ShapeDtypeStruct((N_EDGES, 32), jnp.float32),
                   jax.ShapeDtypeStruct((1, 32), jnp.float32)],
    )(*flat_in)


def _node_kernel(v_prev, v_f, esum_stack, deg_stack, u_f, conv_layers):
    S = esum_stack.shape[0]
    W1, b1 = conv_layers[0]['W'], conv_layers[0]['b'].reshape(1, -1)
    Wa, Wb, Wc = W1[0:32], W1[32:64], W1[64:96]
    rest = _wb(conv_layers[1:])
    nrest = len(conv_layers) - 1

    def body(vp_ref, vf_ref, es_ref, dg_ref, u_ref, Wa_ref, Wb_ref, Wc_ref,
             b1_ref, *refs):
        vout_ref, acc_ref = refs[-2], refs[-1]
        esum = jnp.sum(es_ref[...], axis=0)
        deg = jnp.sum(dg_ref[...], axis=0)[:, 0:1]
        emean = esum / jnp.maximum(deg, 1.0)
        uvec = jnp.dot(u_ref[...], Wc_ref[...],
                       preferred_element_type=jnp.float32)
        h = (jnp.dot(vf_ref[...], Wa_ref[...],
                     preferred_element_type=jnp.float32)
             + jnp.dot(emean, Wb_ref[...], preferred_element_type=jnp.float32)
             + uvec + b1_ref[...])
        h = _sp(h)
        v_new = _chain(h, refs, 0, nrest, True)
        vout_ref[...] = v_new + vp_ref[...]
        i = pl.program_id(0)

        @pl.when(i == 0)
        def _():
            acc_ref[...] = jnp.zeros_like(acc_ref)

        acc_ref[...] += jnp.sum(v_new, axis=0, keepdims=True)

    in_specs = [pl.BlockSpec((NT, 32), lambda i: (i, 0)),
                pl.BlockSpec((NT, 32), lambda i: (i, 0)),
                pl.BlockSpec((S, NT, 32), lambda i: (0, i, 0)),
                pl.BlockSpec((S, NT, 8), lambda i: (0, i, 0)),
                pl.BlockSpec((1, 32), lambda i: (0, 0)),
                pl.BlockSpec(Wa.shape, lambda i: (0, 0)),
                pl.BlockSpec(Wb.shape, lambda i: (0, 0)),
                pl.BlockSpec(Wc.shape, lambda i: (0, 0)),
                pl.BlockSpec(b1.shape, lambda i: (0, 0))]
    for w in rest:
        in_specs.append(pl.BlockSpec(w.shape, lambda i: (0, 0)))
    return pl.pallas_call(
        body, grid=(N_NODES // NT,), in_specs=in_specs,
        out_specs=[pl.BlockSpec((NT, 32), lambda i: (i, 0)),
                   pl.BlockSpec((1, 32), lambda i: (0, 0))],
        out_shape=[jax.ShapeDtypeStruct((N_NODES, 32), jnp.float32),
                   jax.ShapeDtypeStruct((1, 32), jnp.float32)],
    )(v_prev, v_f, esum_stack, deg_stack, u_f, Wa, Wb, Wc, b1, *rest)


def _state_kernel(u_prev, u_f, esum, vsum, conv_layers):
    W1, b1 = conv_layers[0]['W'], conv_layers[0]['b'].reshape(1, -1)
    Wa, Wb, Wc = W1[0:32], W1[32:64], W1[64:96]
    rest = _wb(conv_layers[1:])
    nrest = len(conv_layers) - 1

    def body(up_ref, uf_ref, es_ref, vs_ref, Wa_ref, Wb_ref, Wc_ref, b1_ref,
             *refs):
        out_ref = refs[-1]
        em = es_ref[...] * (1.0 / N_EDGES)
        vm = vs_ref[...] * (1.0 / N_NODES)
        h = (jnp.dot(uf_ref[...], Wa_ref[...],
                     preferred_element_type=jnp.float32)
             + jnp.dot(em, Wb_ref[...], preferred_element_type=jnp.float32)
             + jnp.dot(vm, Wc_ref[...], preferred_element_type=jnp.float32)
             + b1_ref[...])
        h = _sp(h)
        u_new = _chain(h, refs, 0, nrest, True)
        out_ref[...] = u_new + up_ref[...]

    return pl.pallas_call(
        body,
        out_shape=jax.ShapeDtypeStruct((1, 32), jnp.float32),
    )(u_prev, u_f, esum, vsum, Wa, Wb, Wc, b1, *rest)


def _s2s_iter(feat, h, c, q_star, lp, tile):
    rows = feat.shape[0]
    W_ihT = lp['W_ih'].T
    W_hhT = lp['W_hh'].T
    b = (lp['b_ih'] + lp['b_hh']).reshape(1, -1)
    grid = rows // tile

    def body(f_ref, h_ref, c_ref, qs_ref, wih_ref, whh_ref, b_ref,
             hout_ref, cout_ref, qsout_ref, q_ref, m_ref, s_ref, r_ref):
        i = pl.program_id(0)

        @pl.when(i == 0)
        def _():
            gates = (jnp.dot(qs_ref[...], wih_ref[...],
                             preferred_element_type=jnp.float32)
                     + jnp.dot(h_ref[...], whh_ref[...],
                               preferred_element_type=jnp.float32)
                     + b_ref[...])
            ig = gates[:, 0:32]
            fg = gates[:, 32:64]
            gg = gates[:, 64:96]
            og = gates[:, 96:128]
            c_new = (jax.nn.sigmoid(fg) * c_ref[...]
                     + jax.nn.sigmoid(ig) * jnp.tanh(gg))
            h_new = jax.nn.sigmoid(og) * jnp.tanh(c_new)
            hout_ref[...] = h_new
            cout_ref[...] = c_new
            q_ref[...] = h_new
            m_ref[...] = jnp.full_like(m_ref, -jnp.inf)
            s_ref[...] = jnp.zeros_like(s_ref)
            r_ref[...] = jnp.zeros_like(r_ref)

        f = f_ref[...]
        q = q_ref[...]
        scores = jnp.sum(f * q, axis=1, keepdims=True)
        m_old = m_ref[0, 0]
        m_new = jnp.maximum(m_old, jnp.max(scores))
        corr = jnp.exp(m_old - m_new)
        w = jnp.exp(scores - m_new)
        s_ref[...] = s_ref[...] * corr + jnp.sum(w, axis=0, keepdims=True)
        r_ref[...] = r_ref[...] * corr + jnp.sum(w * f, axis=0, keepdims=True)
        m_ref[...] = jnp.full_like(m_ref, m_new)

        @pl.when(i == grid - 1)
        def _():
            r = r_ref[...] / s_ref[0, 0]
            qsout_ref[:, 0:32] = q_ref[...]
            qsout_ref[:, 32:64] = r

    in_specs = [pl.BlockSpec((tile, 32), lambda i: (i, 0)),
                pl.BlockSpec((1, 32), lambda i: (0, 0)),
                pl.BlockSpec((1, 32), lambda i: (0, 0)),
                pl.BlockSpec((1, 64), lambda i: (0, 0)),
                pl.BlockSpec(W_ihT.shape, lambda i: (0, 0)),
                pl.BlockSpec(W_hhT.shape, lambda i: (0, 0)),
                pl.BlockSpec(b.shape, lambda i: (0, 0))]
    out_specs = [pl.BlockSpec((1, 32), lambda i: (0, 0)),
                 pl.BlockSpec((1, 32), lambda i: (0, 0)),
                 pl.BlockSpec((1, 64), lambda i: (0, 0))]
    h2, c2, qs2 = pl.pallas_call(
        body, grid=(grid,), in_specs=in_specs, out_specs=out_specs,
        out_shape=[jax.ShapeDtypeStruct((1, 32), jnp.float32),
                   jax.ShapeDtypeStruct((1, 32), jnp.float32),
                   jax.ShapeDtypeStruct((1, 64), jnp.float32)],
        scratch_shapes=[pltpu.VMEM((1, 32), jnp.float32),
                        pltpu.VMEM((1, 1), jnp.float32),
                        pltpu.VMEM((1, 1), jnp.float32),
                        pltpu.VMEM((1, 32), jnp.float32)],
    )(feat, h, c, q_star, W_ihT, W_hhT, b)
    return h2, c2, qs2


def _set2set(feat, lp, tile):
    h = jnp.zeros((1, 32), jnp.float32)
    c = jnp.zeros((1, 32), jnp.float32)
    q_star = jnp.zeros((1, 64), jnp.float32)
    for _ in range(2):
        h, c, q_star = _s2s_iter(feat, h, c, q_star, lp, tile)
    return q_star


def _out_kernel(qs_node, qs_edge, u, gap_layers, unc_layers):
    Wg1, bg1 = gap_layers[0]['W'], gap_layers[0]['b'].reshape(1, -1)
    Wu1, bu1 = unc_layers[0]['W'], unc_layers[0]['b'].reshape(1, -1)
    g_rest = _wb(gap_layers[1:])
    u_rest = _wb(unc_layers[1:])
    ng, nu = len(gap_layers) - 1, len(unc_layers) - 1

    def head(qn, qe, uu, W1, b1, refs, start, nrest):
        h = (jnp.dot(qn, W1[0:64], preferred_element_type=jnp.float32)
             + jnp.dot(qe, W1[64:128], preferred_element_type=jnp.float32)
             + jnp.dot(uu, W1[128:160], preferred_element_type=jnp.float32)
             + b1)
        h = _sp(h)
        return _chain(h, refs, start, nrest, False)

    def body(qn_ref, qe_ref, u_ref, Wg_ref, bg_ref, Wu_ref, bu_ref, *refs):
        unc_ref, gap_ref = refs[-2], refs[-1]
        qn, qe, uu = qn_ref[...], qe_ref[...], u_ref[...]
        gap_ref[...] = head(qn, qe, uu, Wg_ref[...], bg_ref[...], refs, 0, ng)
        unc_ref[...] = head(qn, qe, uu, Wu_ref[...], bu_ref[...], refs,
                            2 * ng, nu)

    unc, gap = pl.pallas_call(
        body,
        out_shape=[jax.ShapeDtypeStruct((1, 1), jnp.float32),
                   jax.ShapeDtypeStruct((1, 1), jnp.float32)],
    )(qs_node, qs_edge, u, Wg1, bg1, Wu1, bu1, *g_rest, *u_rest)
    return unc, gap


def _gather_nodes(v, idx):
    return v[idx]


def _segment_sums(e_new, dst):
    esum = jax.ops.segment_sum(e_new, dst, num_segments=N_NODES)
    return esum.reshape(1, N_NODES, 32)


def _degree(dst):
    deg = jax.ops.segment_sum(jnp.ones((N_EDGES,), jnp.float32), dst,
                              num_segments=N_NODES)
    dg = jnp.zeros((1, N_NODES, 8), jnp.float32)
    return dg.at[0, :, 0].set(deg)


def kernel(edge_index, node_type, edge_feat, state_feat, params):
    src = edge_index[0]
    dst = edge_index[1]

    v = _node_enc(node_type, params['node_emb'], params['node_enc'])
    e = _rowwise_mlp(edge_feat, params['edge_enc'], ET, True)
    u = _small_mlp(state_feat, params['state_enc'], True)

    deg_stack = _degree(dst)

    for b, bp in enumerate(params['blocks']):
        if bp['edge_func']:
            v_f = _rowwise_mlp(v, bp['node_func'], NT, True)
            u_f = _small_mlp(u, bp['state_func'], True)
            ef_layers = bp['edge_func']
        else:
            v_f = v
            u_f = u
            ef_layers = []
        gs = _gather_nodes(v_f, src)
        gd = _gather_nodes(v_f, dst)
        e_new, e_out, esum_acc = _edge_kernel(e, gs, gd, u_f, ef_layers,
                                              bp['conv_edge'])
        esum_stack = _segment_sums(e_new, dst)
        v_out, vsum_acc = _node_kernel(v, v_f, esum_stack, deg_stack, u_f,
                                       bp['conv_node'])
        u_out = _state_kernel(u, u_f, esum_acc, vsum_acc, bp['conv_state'])
        e, v, u = e_out, v_out, u_out

    qs_node = _set2set(v, params['node_s2s'], NT)
    qs_edge = _set2set(e, params['edge_s2s'], ET)

    unc, gap = _out_kernel(qs_node, qs_edge, u, params['out_gap'],
                           params['out_unc'])
    return unc.reshape(1), gap.reshape(1)
